# baseline (speedup 1.0000x reference)
import functools

import jax
import jax.numpy as jnp
from jax import lax
from jax.experimental import pallas as pl
from jax.experimental.pallas import tpu as pltpu
from jax.experimental.pallas import tpu_sc as plsc

F = 26
B = 4096
D = 32
V = 1000000

NC = 2
NS = 16
NW = NC * NS

CHUNK = B // NW

VBLK = 8192
GRID = -(-V // VBLK)
NROWS = GRID * VBLK * D // 128


def _transpose_body(in_ref, out_ref):
  x = in_ref[...].reshape(D, VBLK)
  for c in range(VBLK // 512):
    xs = jnp.concatenate(
        [x[:, 512 * c + 128 * a:512 * c + 128 * (a + 1)] for a in range(4)],
        axis=0,
    )
    out_ref[128 * c:128 * (c + 1), :] = xs.T


_TRANSPOSE = pl.pallas_call(
    _transpose_body,
    grid=(GRID,),
    in_specs=[pl.BlockSpec((4, 8, VBLK), lambda j: (0, 0, j))],
    out_specs=pl.BlockSpec((VBLK * D // 128, 128), lambda j: (j, 0)),
    out_shape=jax.ShapeDtypeStruct((NROWS, 128), jnp.float32),
)


def _permuted_rows(values):
  v = values
  return (
      (v & ~(VBLK - 1))
      + ((v >> 9) & (VBLK // 512 - 1)) * 512
      + ((v & 127) << 2)
      + ((v >> 7) & 3)
  )


def _build_gather():
  mesh = plsc.VectorSubcoreMesh(core_axis_name="c", subcore_axis_name="s")

  @functools.partial(
      pl.kernel,
      out_type=jax.ShapeDtypeStruct((F, B, D), jnp.float32),
      mesh=mesh,
      scratch_types=[
          pltpu.VMEM((F, CHUNK), jnp.int32),
          pltpu.VMEM((F, CHUNK, D), jnp.float32),
          pltpu.SemaphoreType.DMA,
      ],
      compiler_params=pltpu.CompilerParams(use_tc_tiling_on_sc=False),
  )
  def gather_kernel(tab_hbm, idx_hbm, out_hbm, idx_v, rows_v, sem):
    wid = lax.axis_index("s") * NC + lax.axis_index("c")
    col0 = wid * CHUNK
    pltpu.sync_copy(idx_hbm.at[wid], idx_v)

    def feature_body(f, carry):
      pltpu.async_copy(
          tab_hbm.at[idx_v.at[f]], rows_v.at[f], sem
      ).wait()
      pltpu.sync_copy(
          rows_v.at[f], out_hbm.at[f, pl.ds(col0, CHUNK), :]
      )
      return carry

    lax.fori_loop(0, F, feature_body, 0, unroll=False)

  return gather_kernel


_GATHER = _build_gather()


def kernel(table, values, lengths):
  tab3 = table.T.reshape(4, 8, V)
  tablin = _TRANSPOSE(tab3)
  tab_flat = tablin.reshape(GRID * VBLK, D)
  idx = _permuted_rows(values).reshape(F, NW, CHUNK).transpose(1, 0, 2)
  split_embeddings = _GATHER(tab_flat, idx)
  split_lengths = lengths.reshape(F, B)
  reduce_lengths = split_lengths.sum(axis=1)
  offsets = jnp.concatenate([
      jnp.zeros((1,), dtype=reduce_lengths.dtype),
      jnp.cumsum(reduce_lengths),
  ])
  return split_embeddings, split_lengths, offsets
